# baseline (speedup 1.0000x reference)
import functools

import jax
import jax.numpy as jnp
from jax import lax
from jax.experimental import pallas as pl
from jax.experimental.pallas import tpu as pltpu
from jax.experimental.pallas import tpu_sc as plsc

_VOCAB = 1000000
_DIM = 64
_BATCH = 16384

_INFO = plsc.get_sparse_core_info()
_NC = _INFO.num_cores
_NS = _INFO.num_subcores
_L = _INFO.num_lanes
_NW = _NC * _NS
_BPW = _BATCH // _NW


def _glove_kernel(word_ids_hbm, ctx_ids_hbm, wemb_hbm, cemb_hbm,
                  wbias_hbm, cbias_hbm, out_hbm,
                  widx_v, cidx_v, wrows_v, crows_v, wb_v, cb_v, out_v, sem):
    wid = lax.axis_index("s") * _NC + lax.axis_index("c")
    base = wid * _BPW

    pltpu.sync_copy(word_ids_hbm.at[pl.ds(base, _BPW)], widx_v)
    pltpu.sync_copy(ctx_ids_hbm.at[pl.ds(base, _BPW)], cidx_v)

    cp_w = pltpu.async_copy(wemb_hbm.at[widx_v], wrows_v, sem)
    cp_c = pltpu.async_copy(cemb_hbm.at[cidx_v], crows_v, sem)
    cp_wb = pltpu.async_copy(wbias_hbm.at[widx_v], wb_v, sem)
    cp_cb = pltpu.async_copy(cbias_hbm.at[cidx_v], cb_v, sem)
    cp_w.wait()
    cp_c.wait()
    cp_wb.wait()
    cp_cb.wait()

    def chunk_body(i, carry):
        rows = lax.iota(jnp.int32, _L) + i * _L
        acc = wb_v[pl.ds(i * _L, _L)] + cb_v[pl.ds(i * _L, _L)]
        for d in range(_DIM):
            col = jnp.full((_L,), d, jnp.int32)
            wv = plsc.load_gather(wrows_v, [rows, col])
            cv = plsc.load_gather(crows_v, [rows, col])
            acc = acc + wv * cv
        out_v[pl.ds(i * _L, _L)] = acc
        return carry

    lax.fori_loop(0, _BPW // _L, chunk_body, 0)

    pltpu.sync_copy(out_v, out_hbm.at[pl.ds(base, _BPW)])


@jax.jit
def kernel(word_ids, context_ids, word_embeddings, context_embeddings,
           word_biases, context_biases):
    mesh = plsc.VectorSubcoreMesh(core_axis_name="c", subcore_axis_name="s")
    run = functools.partial(
        pl.kernel,
        mesh=mesh,
        compiler_params=pltpu.CompilerParams(
            needs_layout_passes=False, use_tc_tiling_on_sc=False),
        out_type=jax.ShapeDtypeStruct((_BATCH,), jnp.float32),
        scratch_types=[
            pltpu.VMEM((_BPW,), jnp.int32),
            pltpu.VMEM((_BPW,), jnp.int32),
            pltpu.VMEM((_BPW, _DIM), jnp.float32),
            pltpu.VMEM((_BPW, _DIM), jnp.float32),
            pltpu.VMEM((_BPW,), jnp.float32),
            pltpu.VMEM((_BPW,), jnp.float32),
            pltpu.VMEM((_BPW,), jnp.float32),
            pltpu.SemaphoreType.DMA,
        ],
    )(_glove_kernel)
    return run(word_ids.astype(jnp.int32), context_ids.astype(jnp.int32),
               word_embeddings, context_embeddings,
               word_biases.reshape(_VOCAB), context_biases.reshape(_VOCAB))
